# Initial kernel scaffold; baseline (speedup 1.0000x reference)
import jax
import jax.numpy as jnp
from jax.experimental import pallas as pl


def kernel(input_ids, embedding):
    raise NotImplementedError("write your pallas kernel here")


# speedup vs baseline: 5.7929x; 5.7929x over previous
import functools

import jax
import jax.numpy as jnp
from jax import lax
from jax.experimental import pallas as pl
from jax.experimental.pallas import tpu as pltpu
from jax.experimental.pallas import tpu_sc as plsc

VOCAB_SIZE = 100000
P = 16
ALPHA = 8
D = P * ALPHA
B = 1024
L = 200

NC = 2
NS = 16
NW = NC * NS

N_ROWS = B * L
ROWS_PER_W = N_ROWS // NW
CHUNK = 128
N_CHUNKS = ROWS_PER_W // CHUNK


@functools.partial(
    pl.kernel,
    out_type=jax.ShapeDtypeStruct((N_ROWS, D), jnp.float32),
    mesh=plsc.VectorSubcoreMesh(core_axis_name="c", subcore_axis_name="s"),
    scratch_types=[
        pltpu.VMEM((N_CHUNKS, CHUNK), jnp.int32),
        pltpu.VMEM((CHUNK, D), jnp.float32),
        pltpu.SemaphoreType.DMA,
    ],
)
def _gather_kernel(idx_hbm, table_hbm, out_hbm, idx_v, rows_v, sem):
    wid = lax.axis_index("s") * NC + lax.axis_index("c")
    base = wid * ROWS_PER_W
    pltpu.sync_copy(idx_hbm.at[wid], idx_v)

    def body(j, carry):
        pltpu.async_copy(table_hbm.at[idx_v.at[j]], rows_v, sem).wait()
        pltpu.sync_copy(rows_v, out_hbm.at[pl.ds(base + j * CHUNK, CHUNK)])
        return carry

    lax.fori_loop(0, N_CHUNKS, body, 0)


def kernel(input_ids, embedding):
    idx = input_ids.reshape(NW, N_CHUNKS, CHUNK).astype(jnp.int32)
    out = _gather_kernel(idx, embedding)
    return out.reshape(B, L, D)


# speedup vs baseline: 7.9587x; 1.3739x over previous
import functools

import jax
import jax.numpy as jnp
from jax import lax
from jax.experimental import pallas as pl
from jax.experimental.pallas import tpu as pltpu
from jax.experimental.pallas import tpu_sc as plsc

VOCAB_SIZE = 100000
P = 16
ALPHA = 8
D = P * ALPHA
B = 1024
L = 200

NC = 2
NS = 16
NW = NC * NS

N_ROWS = B * L
ROWS_PER_W = N_ROWS // NW
CHUNK = 128
N_CHUNKS = ROWS_PER_W // CHUNK
NBUF = 1 + 1 + 1 + 1


@functools.partial(
    pl.kernel,
    out_type=jax.ShapeDtypeStruct((N_ROWS, D), jnp.float32),
    mesh=plsc.VectorSubcoreMesh(core_axis_name="c", subcore_axis_name="s"),
    scratch_types=[
        pltpu.VMEM((N_CHUNKS, CHUNK), jnp.int32),
        [pltpu.VMEM((CHUNK, D), jnp.float32) for _ in range(NBUF)],
        [pltpu.SemaphoreType.DMA for _ in range(NBUF)],
        [pltpu.SemaphoreType.DMA for _ in range(NBUF)],
    ],
)
def _gather_kernel(idx_hbm, table_hbm, out_hbm, idx_v, bufs, gsems, osems):
    wid = lax.axis_index("s") * NC + lax.axis_index("c")
    base = wid * ROWS_PER_W
    pltpu.sync_copy(idx_hbm.at[wid], idx_v)

    def start_gather(j, pos):
        pltpu.async_copy(table_hbm.at[idx_v.at[j]], bufs[pos], gsems[pos])

    def wait_gather(j, pos):
        pltpu.make_async_copy(table_hbm.at[idx_v.at[j]], bufs[pos],
                              gsems[pos]).wait()

    def start_out(j, pos):
        pltpu.async_copy(bufs[pos], out_hbm.at[pl.ds(base + j * CHUNK, CHUNK)],
                         osems[pos])

    def wait_out(j, pos):
        pltpu.make_async_copy(bufs[pos],
                              out_hbm.at[pl.ds(base + j * CHUNK, CHUNK)],
                              osems[pos]).wait()

    start_gather(0, 0)
    start_gather(1, 1)
    wait_gather(0, 0)
    start_out(0, 0)
    start_gather(2, 2)
    wait_gather(1, 1)
    start_out(1, 1)
    start_gather(3, 3)

    def body(i, carry):
        j0 = 2 + i * 4
        for off in range(4):
            j = j0 + off
            pos = (2 + off) % 4
            wait_gather(j, pos)
            start_out(j, pos)
            npos = (pos + 2) % 4
            wait_out(j - 2, npos)
            start_gather(j + 2, npos)
        return carry

    lax.fori_loop(0, (N_CHUNKS - 6) // 4, body, 0)

    wait_gather(46, 2)
    start_out(46, 2)
    wait_out(44, 0)
    start_gather(48, 0)
    wait_gather(47, 3)
    start_out(47, 3)
    wait_out(45, 1)
    start_gather(49, 1)
    wait_gather(48, 0)
    start_out(48, 0)
    wait_gather(49, 1)
    start_out(49, 1)
    wait_out(46, 2)
    wait_out(47, 3)
    wait_out(48, 0)
    wait_out(49, 1)


def kernel(input_ids, embedding):
    idx = input_ids.reshape(NW, N_CHUNKS, CHUNK).astype(jnp.int32)
    out = _gather_kernel(idx, embedding)
    return out.reshape(B, L, D)
